# baseline (speedup 1.0000x reference)
import functools

import jax
import jax.numpy as jnp
from jax import lax
from jax.experimental import pallas as pl
from jax.experimental.pallas import tpu as pltpu
from jax.experimental.pallas import tpu_sc as plsc

N = 100000
D = 128
CD = 9000
C0 = 10000
MAX_DEG = 10
B = 40
NCH = CD // B
GMAX = 120


def _make_call(dlist, with_deg0):
    emax = B * max(dlist)
    mesh = plsc.VectorSubcoreMesh(core_axis_name="c", subcore_axis_name="s")
    nw = mesh.num_cores * mesh.num_subcores

    @functools.partial(
        pl.kernel,
        out_type=(),
        mesh=mesh,
        scratch_types=[
            pltpu.VMEM((2 * emax,), jnp.int32),
            pltpu.VMEM((2, emax, D), jnp.float32),
            pltpu.VMEM((2, B, D), jnp.float32),
            pltpu.VMEM((2, B, D), jnp.float32),
            pltpu.SemaphoreType.DMA,
            pltpu.SemaphoreType.DMA,
            pltpu.SemaphoreType.DMA,
            pltpu.SemaphoreType.DMA,
            pltpu.SemaphoreType.DMA,
            pltpu.SemaphoreType.DMA,
            pltpu.SemaphoreType.DMA,
            pltpu.SemaphoreType.DMA,
            pltpu.SemaphoreType.DMA,
        ],
    )
    def k(atoms_hbm, *rest):
        adjs = rest[:len(dlist)]
        out_hbm = rest[len(dlist)]
        (idx_v, gath_v, self_v, outb_v,
         gsem0, gsem1, isem0, isem1, fsem0, fsem1, ssem0, ssem1,
         dsem) = rest[len(dlist) + 1:]
        gsems = (gsem0, gsem1)
        isems = (isem0, isem1)
        fsems = (fsem0, fsem1)
        ssems = (ssem0, ssem1)
        wid = lax.axis_index("s") * mesh.num_cores + lax.axis_index("c")

        if with_deg0:
            r0a = pl.multiple_of(312 * wid + 8 * jnp.minimum(wid, 2), 8)

            @pl.when(wid < 2)
            def _():
                pltpu.async_copy(atoms_hbm.at[pl.ds(r0a, 320)],
                                 out_hbm.at[pl.ds(r0a, 320)], dsem)

            @pl.when(wid >= 2)
            def _():
                pltpu.async_copy(atoms_hbm.at[pl.ds(r0a, 312)],
                                 out_hbm.at[pl.ds(r0a, 312)], dsem)

        for di, d in enumerate(dlist):
            ecount = B * d
            edges_hbm = adjs[di]
            start = C0 + CD * (d - 1)
            pieces = [(off, min(GMAX, ecount - off))
                      for off in range(0, ecount, GMAX)]
            rot = (13 * d) % nw
            cw = (wid + rot) % nw
            nch_w = (NCH - cw + nw - 1) // nw

            def issue_idx(kk, b, ecount=ecount, edges_hbm=edges_hbm, cw=cw):
                c = cw + kk * nw
                pltpu.async_copy(
                    edges_hbm.at[pl.ds(c * ecount, ecount)],
                    idx_v.at[pl.ds(b * emax, ecount)], isems[b])

            def wait_idx(b, ecount=ecount, edges_hbm=edges_hbm):
                pltpu.make_async_copy(
                    edges_hbm.at[pl.ds(0, ecount)],
                    idx_v.at[pl.ds(b * emax, ecount)], isems[b]).wait()

            def issue_self(kk, b, start=start, cw=cw):
                c = cw + kk * nw
                pltpu.async_copy(atoms_hbm.at[pl.ds(start + c * B, B)],
                                 self_v.at[b], fsems[b])

            def wait_self(b):
                pltpu.make_async_copy(atoms_hbm.at[pl.ds(0, B)],
                                      self_v.at[b], fsems[b]).wait()

            def issue_gath(b, pieces=pieces):
                for off, sz in pieces:
                    pltpu.async_copy(
                        atoms_hbm.at[idx_v.at[pl.ds(b * emax + off, sz)]],
                        gath_v.at[b, pl.ds(off, sz)], gsems[b])

            def wait_gath(b, ecount=ecount):
                pltpu.make_async_copy(
                    atoms_hbm.at[pl.ds(0, ecount)],
                    gath_v.at[b, pl.ds(0, ecount)], gsems[b]).wait()

            def compute(kk, b, d=d, start=start, cw=cw):
                c = cw + kk * nw

                @pl.when(kk >= 2)
                def _():
                    pltpu.make_async_copy(
                        outb_v.at[b], out_hbm.at[pl.ds(0, B)], ssems[b]).wait()

                wait_self(b)

                @pl.loop(0, B)
                def row_body(i):
                    base = i * d
                    accs = [self_v[b, i, pl.ds(s * 16, 16)]
                            for s in range(D // 16)]
                    for j in range(d):
                        for s in range(D // 16):
                            accs[s] = jnp.maximum(
                                accs[s],
                                gath_v[b, base + j, pl.ds(s * 16, 16)])
                    for s in range(D // 16):
                        outb_v[b, i, pl.ds(s * 16, 16)] = accs[s]

                pltpu.async_copy(outb_v.at[b],
                                 out_hbm.at[pl.ds(start + c * B, B)], ssems[b])

            issue_idx(0, 0)
            issue_idx(1, 1)
            issue_self(0, 0)
            issue_self(1, 1)
            wait_idx(0)
            issue_gath(0)

            @pl.loop(0, nch_w)
            def chunk_body(kk, nch_w=nch_w, issue_idx=issue_idx,
                           wait_idx=wait_idx, issue_gath=issue_gath,
                           wait_gath=wait_gath, issue_self=issue_self,
                           compute=compute):
                for b in (0, 1):
                    @pl.when(kk % 2 == b)
                    def _(b=b, kk=kk):
                        wait_gath(b)

                        @pl.when(kk + 2 < nch_w)
                        def _():
                            issue_idx(kk + 2, b)

                        @pl.when(kk + 1 < nch_w)
                        def _(b=b):
                            wait_idx(1 - b)
                            issue_gath(1 - b)

                        compute(kk, b)

                        @pl.when(kk + 2 < nch_w)
                        def _():
                            issue_self(kk + 2, b)

            for b in (0, 1):
                pltpu.make_async_copy(
                    outb_v.at[b], out_hbm.at[pl.ds(0, B)], ssems[b]).wait()

        if with_deg0:
            @pl.when(wid < 2)
            def _():
                pltpu.make_async_copy(atoms_hbm.at[pl.ds(0, 320)],
                                      out_hbm.at[pl.ds(0, 320)], dsem).wait()

            @pl.when(wid >= 2)
            def _():
                pltpu.make_async_copy(atoms_hbm.at[pl.ds(0, 312)],
                                      out_hbm.at[pl.ds(0, 312)], dsem).wait()

    return k


def kernel(atoms, deg_slice, membership, deg_adj_1, deg_adj_2, deg_adj_3,
           deg_adj_4, deg_adj_5, deg_adj_6, deg_adj_7, deg_adj_8,
           deg_adj_9, deg_adj_10):
    del deg_slice, membership
    small = [a.reshape(-1) for a in (deg_adj_1, deg_adj_2, deg_adj_3)]
    big = [a.reshape(-1) for a in
           (deg_adj_4, deg_adj_5, deg_adj_6, deg_adj_7, deg_adj_8,
            deg_adj_9, deg_adj_10)]
    out_ref = jax.new_ref(lax.empty((N, D), jnp.float32))
    _make_call((1, 2, 3), True)(atoms, *small, out_ref)
    _make_call((4, 5, 6, 7, 8, 9, 10), False)(atoms, *big, out_ref)
    return out_ref[...]


# speedup vs baseline: 1.3373x; 1.3373x over previous
import functools

import jax
import jax.numpy as jnp
from jax import lax
from jax.experimental import pallas as pl
from jax.experimental.pallas import tpu as pltpu
from jax.experimental.pallas import tpu_sc as plsc

N = 100000
D = 128
CD = 9000
C0 = 10000
MAX_DEG = 10
B = 40
NCH = CD // B
GMAX = 120
EMAX = B * MAX_DEG


def _pool(atoms, adj_flats):
    mesh = plsc.VectorSubcoreMesh(core_axis_name="c", subcore_axis_name="s")
    nw = mesh.num_cores * mesh.num_subcores

    @functools.partial(
        pl.kernel,
        out_type=jax.ShapeDtypeStruct((N, D), jnp.float32),
        mesh=mesh,
        scratch_types=[
            pltpu.VMEM((2 * EMAX,), jnp.int32),
            pltpu.VMEM((2, EMAX, D), jnp.float32),
            pltpu.VMEM((2, B, D), jnp.float32),
            pltpu.VMEM((2, B, D), jnp.float32),
            pltpu.SemaphoreType.DMA,
            pltpu.SemaphoreType.DMA,
            pltpu.SemaphoreType.DMA,
            pltpu.SemaphoreType.DMA,
            pltpu.SemaphoreType.DMA,
            pltpu.SemaphoreType.DMA,
            pltpu.SemaphoreType.DMA,
            pltpu.SemaphoreType.DMA,
            pltpu.SemaphoreType.DMA,
        ],
    )
    def k(atoms_hbm, a1, a2, a3, a4, a5, a6, a7, a8, a9, a10, out_hbm,
          idx_v, gath_v, self_v, outb_v,
          gsem0, gsem1, isem0, isem1, fsem0, fsem1, ssem0, ssem1, dsem):
        adjs = (a1, a2, a3, a4, a5, a6, a7, a8, a9, a10)
        gsems = (gsem0, gsem1)
        isems = (isem0, isem1)
        fsems = (fsem0, fsem1)
        ssems = (ssem0, ssem1)
        wid = lax.axis_index("s") * mesh.num_cores + lax.axis_index("c")

        r0a = pl.multiple_of(312 * wid + 8 * jnp.minimum(wid, 2), 8)

        @pl.when(wid < 2)
        def _():
            pltpu.async_copy(atoms_hbm.at[pl.ds(r0a, 320)],
                             out_hbm.at[pl.ds(r0a, 320)], dsem)

        @pl.when(wid >= 2)
        def _():
            pltpu.async_copy(atoms_hbm.at[pl.ds(r0a, 312)],
                             out_hbm.at[pl.ds(r0a, 312)], dsem)

        for d in range(1, MAX_DEG + 1):
            ecount = B * d
            edges_hbm = adjs[d - 1]
            start = C0 + CD * (d - 1)
            pieces = [(off, min(GMAX, ecount - off))
                      for off in range(0, ecount, GMAX)]
            rot = (13 * d) % nw
            cw = (wid + rot) % nw
            nch_w = (NCH - cw + nw - 1) // nw

            def issue_idx(kk, b, ecount=ecount, edges_hbm=edges_hbm, cw=cw):
                c = cw + kk * nw
                pltpu.async_copy(
                    edges_hbm.at[pl.ds(c * ecount, ecount)],
                    idx_v.at[pl.ds(b * EMAX, ecount)], isems[b])

            def wait_idx(b, ecount=ecount, edges_hbm=edges_hbm):
                pltpu.make_async_copy(
                    edges_hbm.at[pl.ds(0, ecount)],
                    idx_v.at[pl.ds(b * EMAX, ecount)], isems[b]).wait()

            def issue_self(kk, b, start=start, cw=cw):
                c = cw + kk * nw
                pltpu.async_copy(atoms_hbm.at[pl.ds(start + c * B, B)],
                                 self_v.at[b], fsems[b])

            def wait_self(b):
                pltpu.make_async_copy(atoms_hbm.at[pl.ds(0, B)],
                                      self_v.at[b], fsems[b]).wait()

            def issue_gath(b, pieces=pieces):
                for off, sz in pieces:
                    pltpu.async_copy(
                        atoms_hbm.at[idx_v.at[pl.ds(b * EMAX + off, sz)]],
                        gath_v.at[b, pl.ds(off, sz)], gsems[b])

            def wait_gath(b, ecount=ecount):
                pltpu.make_async_copy(
                    atoms_hbm.at[pl.ds(0, ecount)],
                    gath_v.at[b, pl.ds(0, ecount)], gsems[b]).wait()

            def compute(kk, b, d=d, start=start, cw=cw):
                c = cw + kk * nw

                @pl.when(kk >= 2)
                def _():
                    pltpu.make_async_copy(
                        outb_v.at[b], out_hbm.at[pl.ds(0, B)], ssems[b]).wait()

                wait_self(b)

                @pl.loop(0, B, unroll=2)
                def row_body(i):
                    base = i * d
                    accs = [self_v[b, i, pl.ds(s * 16, 16)]
                            for s in range(D // 16)]
                    for j in range(d):
                        for s in range(D // 16):
                            accs[s] = jnp.maximum(
                                accs[s], gath_v[b, base + j, pl.ds(s * 16, 16)])
                    for s in range(D // 16):
                        outb_v[b, i, pl.ds(s * 16, 16)] = accs[s]

                pltpu.async_copy(outb_v.at[b],
                                 out_hbm.at[pl.ds(start + c * B, B)], ssems[b])

            issue_idx(0, 0)
            issue_idx(1, 1)
            issue_self(0, 0)
            issue_self(1, 1)
            wait_idx(0)
            issue_gath(0)

            @pl.loop(0, nch_w)
            def chunk_body(kk, nch_w=nch_w, issue_idx=issue_idx,
                           wait_idx=wait_idx, issue_gath=issue_gath,
                           wait_gath=wait_gath, issue_self=issue_self,
                           compute=compute):
                for b in (0, 1):
                    @pl.when(kk % 2 == b)
                    def _(b=b, kk=kk):
                        wait_gath(b)

                        @pl.when(kk + 2 < nch_w)
                        def _():
                            issue_idx(kk + 2, b)

                        @pl.when(kk + 1 < nch_w)
                        def _(b=b):
                            wait_idx(1 - b)
                            issue_gath(1 - b)

                        compute(kk, b)

                        @pl.when(kk + 2 < nch_w)
                        def _():
                            issue_self(kk + 2, b)

            for b in (0, 1):
                pltpu.make_async_copy(
                    outb_v.at[b], out_hbm.at[pl.ds(0, B)], ssems[b]).wait()

        @pl.when(wid < 2)
        def _():
            pltpu.make_async_copy(atoms_hbm.at[pl.ds(0, 320)],
                                  out_hbm.at[pl.ds(0, 320)], dsem).wait()

        @pl.when(wid >= 2)
        def _():
            pltpu.make_async_copy(atoms_hbm.at[pl.ds(0, 312)],
                                  out_hbm.at[pl.ds(0, 312)], dsem).wait()

    return k(atoms, *adj_flats)


def kernel(atoms, deg_slice, membership, deg_adj_1, deg_adj_2, deg_adj_3,
           deg_adj_4, deg_adj_5, deg_adj_6, deg_adj_7, deg_adj_8,
           deg_adj_9, deg_adj_10):
    del deg_slice, membership
    adj_flats = [a.reshape(-1) for a in
                 (deg_adj_1, deg_adj_2, deg_adj_3, deg_adj_4, deg_adj_5,
                  deg_adj_6, deg_adj_7, deg_adj_8, deg_adj_9, deg_adj_10)]
    return _pool(atoms, adj_flats)


# speedup vs baseline: 1.3611x; 1.0178x over previous
import functools

import jax
import jax.numpy as jnp
from jax import lax
from jax.experimental import pallas as pl
from jax.experimental.pallas import tpu as pltpu
from jax.experimental.pallas import tpu_sc as plsc

N = 100000
D = 128
CD = 9000
C0 = 10000
MAX_DEG = 10
B = 40
NCH = CD // B
GMAX = 120
EMAX = B * MAX_DEG


def _pool(atoms, adj_flats):
    mesh = plsc.VectorSubcoreMesh(core_axis_name="c", subcore_axis_name="s")
    nw = mesh.num_cores * mesh.num_subcores

    @functools.partial(
        pl.kernel,
        out_type=jax.ShapeDtypeStruct((N, D), jnp.float32),
        mesh=mesh,
        scratch_types=[
            pltpu.VMEM((2 * EMAX,), jnp.int32),
            pltpu.VMEM((20 * (MAX_DEG + 1) * MAX_DEG,), jnp.int32),
            pltpu.VMEM((2, EMAX, D), jnp.float32),
            pltpu.VMEM((2, B, D), jnp.float32),
            pltpu.VMEM((2, B, D), jnp.float32),
            pltpu.SemaphoreType.DMA,
            pltpu.SemaphoreType.DMA,
            pltpu.SemaphoreType.DMA,
            pltpu.SemaphoreType.DMA,
            pltpu.SemaphoreType.DMA,
            pltpu.SemaphoreType.DMA,
            pltpu.SemaphoreType.DMA,
            pltpu.SemaphoreType.DMA,
            pltpu.SemaphoreType.DMA,
            pltpu.SemaphoreType.DMA,
        ],
    )
    def k(atoms_hbm, a1, a2, a3, a4, a5, a6, a7, a8, a9, a10, out_hbm,
          idx_v, pidx_v, gath_v, self_v, outb_v,
          gsem0, gsem1, isem0, isem1, fsem0, fsem1, ssem0, ssem1, dsem,
          prosem):
        adjs = (a1, a2, a3, a4, a5, a6, a7, a8, a9, a10)
        gsems = (gsem0, gsem1)
        isems = (isem0, isem1)
        fsems = (fsem0, fsem1)
        ssems = (ssem0, ssem1)
        wid = lax.axis_index("s") * mesh.num_cores + lax.axis_index("c")

        r0a = pl.multiple_of(312 * wid + 8 * jnp.minimum(wid, 2), 8)

        @pl.when(wid < 2)
        def _():
            pltpu.async_copy(atoms_hbm.at[pl.ds(r0a, 320)],
                             out_hbm.at[pl.ds(r0a, 320)], dsem)

        @pl.when(wid >= 2)
        def _():
            pltpu.async_copy(atoms_hbm.at[pl.ds(r0a, 312)],
                             out_hbm.at[pl.ds(r0a, 312)], dsem)

        for d in range(1, MAX_DEG + 1):
            ec = B * d
            cw0 = (wid + (13 * d) % nw) % nw
            pltpu.async_copy(adjs[d - 1].at[pl.ds(cw0 * ec, ec)],
                             pidx_v.at[pl.ds(20 * d * (d - 1), ec)], prosem)
        ptotal = 20 * MAX_DEG * (MAX_DEG + 1)

        for d in range(1, MAX_DEG + 1):
            ecount = B * d
            edges_hbm = adjs[d - 1]
            start = C0 + CD * (d - 1)
            pieces = [(0, ecount)]
            rot = (13 * d) % nw
            cw = (wid + rot) % nw
            nch_w = (NCH - cw + nw - 1) // nw

            def issue_idx(kk, b, ecount=ecount, edges_hbm=edges_hbm, cw=cw):
                c = cw + kk * nw
                pltpu.async_copy(
                    edges_hbm.at[pl.ds(c * ecount, ecount)],
                    idx_v.at[pl.ds(b * EMAX, ecount)], isems[b])

            def wait_idx(b, ecount=ecount, edges_hbm=edges_hbm):
                pltpu.make_async_copy(
                    edges_hbm.at[pl.ds(0, ecount)],
                    idx_v.at[pl.ds(b * EMAX, ecount)], isems[b]).wait()

            def issue_self(kk, b, start=start, cw=cw):
                c = cw + kk * nw
                pltpu.async_copy(atoms_hbm.at[pl.ds(start + c * B, B)],
                                 self_v.at[b], fsems[b])

            def wait_self(b):
                pltpu.make_async_copy(atoms_hbm.at[pl.ds(0, B)],
                                      self_v.at[b], fsems[b]).wait()

            def issue_gath(b, pieces=pieces):
                for off, sz in pieces:
                    pltpu.async_copy(
                        atoms_hbm.at[idx_v.at[pl.ds(b * EMAX + off, sz)]],
                        gath_v.at[b, pl.ds(off, sz)], gsems[b])

            def wait_gath(b, ecount=ecount):
                pltpu.make_async_copy(
                    atoms_hbm.at[pl.ds(0, ecount)],
                    gath_v.at[b, pl.ds(0, ecount)], gsems[b]).wait()

            def compute(kk, b, d=d, start=start, cw=cw):
                c = cw + kk * nw

                @pl.when(kk >= 2)
                def _():
                    pltpu.make_async_copy(
                        outb_v.at[b], out_hbm.at[pl.ds(0, B)], ssems[b]).wait()

                wait_self(b)

                @pl.loop(0, B)
                def row_body(i):
                    base = i * d
                    accs = [self_v[b, i, pl.ds(s * 16, 16)]
                            for s in range(D // 16)]
                    for j in range(d):
                        for s in range(D // 16):
                            accs[s] = jnp.maximum(
                                accs[s], gath_v[b, base + j, pl.ds(s * 16, 16)])
                    for s in range(D // 16):
                        outb_v[b, i, pl.ds(s * 16, 16)] = accs[s]

                pltpu.async_copy(outb_v.at[b],
                                 out_hbm.at[pl.ds(start + c * B, B)], ssems[b])

            issue_idx(1, 1)
            issue_self(0, 0)
            issue_self(1, 1)
            if d == 1:
                pltpu.make_async_copy(
                    a1.at[pl.ds(0, ptotal)],
                    pidx_v.at[pl.ds(0, ptotal)], prosem).wait()
            pltpu.async_copy(
                atoms_hbm.at[pidx_v.at[pl.ds(20 * d * (d - 1), ecount)]],
                gath_v.at[0, pl.ds(0, ecount)], gsems[0])

            @pl.loop(0, nch_w)
            def chunk_body(kk, nch_w=nch_w, issue_idx=issue_idx,
                           wait_idx=wait_idx, issue_gath=issue_gath,
                           wait_gath=wait_gath, issue_self=issue_self,
                           compute=compute):
                for b in (0, 1):
                    @pl.when(kk % 2 == b)
                    def _(b=b, kk=kk):
                        wait_gath(b)

                        @pl.when(kk + 2 < nch_w)
                        def _():
                            issue_idx(kk + 2, b)

                        @pl.when(kk + 1 < nch_w)
                        def _(b=b):
                            wait_idx(1 - b)
                            issue_gath(1 - b)

                        compute(kk, b)

                        @pl.when(kk + 2 < nch_w)
                        def _():
                            issue_self(kk + 2, b)

            for b in (0, 1):
                pltpu.make_async_copy(
                    outb_v.at[b], out_hbm.at[pl.ds(0, B)], ssems[b]).wait()

        @pl.when(wid < 2)
        def _():
            pltpu.make_async_copy(atoms_hbm.at[pl.ds(0, 320)],
                                  out_hbm.at[pl.ds(0, 320)], dsem).wait()

        @pl.when(wid >= 2)
        def _():
            pltpu.make_async_copy(atoms_hbm.at[pl.ds(0, 312)],
                                  out_hbm.at[pl.ds(0, 312)], dsem).wait()

    return k(atoms, *adj_flats)


def kernel(atoms, deg_slice, membership, deg_adj_1, deg_adj_2, deg_adj_3,
           deg_adj_4, deg_adj_5, deg_adj_6, deg_adj_7, deg_adj_8,
           deg_adj_9, deg_adj_10):
    del deg_slice, membership
    adj_flats = [a.reshape(-1) for a in
                 (deg_adj_1, deg_adj_2, deg_adj_3, deg_adj_4, deg_adj_5,
                  deg_adj_6, deg_adj_7, deg_adj_8, deg_adj_9, deg_adj_10)]
    return _pool(atoms, adj_flats)


# speedup vs baseline: 1.3615x; 1.0003x over previous
import functools

import jax
import jax.numpy as jnp
from jax import lax
from jax.experimental import pallas as pl
from jax.experimental.pallas import tpu as pltpu
from jax.experimental.pallas import tpu_sc as plsc

N = 100000
D = 128
CD = 9000
C0 = 10000
MAX_DEG = 10
B = 40
NCH = CD // B
EMAX = B * MAX_DEG


def _pool(atoms, adj_flats):
    mesh = plsc.VectorSubcoreMesh(core_axis_name="c", subcore_axis_name="s")
    nw = mesh.num_cores * mesh.num_subcores

    @functools.partial(
        pl.kernel,
        out_type=jax.ShapeDtypeStruct((N, D), jnp.float32),
        mesh=mesh,
        scratch_types=[
            pltpu.VMEM((2 * EMAX,), jnp.int32),
            pltpu.VMEM((20 * (MAX_DEG + 1) * MAX_DEG,), jnp.int32),
            pltpu.VMEM((2, EMAX, D), jnp.float32),
            pltpu.VMEM((2, B, D), jnp.float32),
            pltpu.VMEM((2, B, D), jnp.float32),
            pltpu.SemaphoreType.DMA,
            pltpu.SemaphoreType.DMA,
            pltpu.SemaphoreType.DMA,
            pltpu.SemaphoreType.DMA,
            pltpu.SemaphoreType.DMA,
            pltpu.SemaphoreType.DMA,
            pltpu.SemaphoreType.DMA,
            pltpu.SemaphoreType.DMA,
            pltpu.SemaphoreType.DMA,
            pltpu.SemaphoreType.DMA,
        ],
    )
    def k(atoms_hbm, a1, a2, a3, a4, a5, a6, a7, a8, a9, a10, out_hbm,
          idx_v, pidx_v, gath_v, self_v, outb_v,
          gsem0, gsem1, isem0, isem1, fsem0, fsem1, ssem0, ssem1, dsem,
          prosem):
        adjs = (a1, a2, a3, a4, a5, a6, a7, a8, a9, a10)
        gsems = (gsem0, gsem1)
        isems = (isem0, isem1)
        fsems = (fsem0, fsem1)
        ssems = (ssem0, ssem1)
        wid = lax.axis_index("s") * mesh.num_cores + lax.axis_index("c")

        r0a = pl.multiple_of(312 * wid + 8 * jnp.minimum(wid, 2), 8)

        @pl.when(wid < 2)
        def _():
            pltpu.async_copy(atoms_hbm.at[pl.ds(r0a, 320)],
                             out_hbm.at[pl.ds(r0a, 320)], dsem)

        @pl.when(wid >= 2)
        def _():
            pltpu.async_copy(atoms_hbm.at[pl.ds(r0a, 312)],
                             out_hbm.at[pl.ds(r0a, 312)], dsem)

        for d in range(1, MAX_DEG + 1):
            ec = B * d
            cw0 = (wid + (13 * d) % nw) % nw
            pltpu.async_copy(adjs[d - 1].at[pl.ds(cw0 * ec, ec)],
                             pidx_v.at[pl.ds(20 * d * (d - 1), ec)], prosem)
        ptotal = 20 * MAX_DEG * (MAX_DEG + 1)

        for d in range(1, MAX_DEG + 1):
            ecount = B * d
            edges_hbm = adjs[d - 1]
            start = C0 + CD * (d - 1)
            pieces = [(0, ecount)]
            rot = (13 * d) % nw
            cw = (wid + rot) % nw
            nch_w = (NCH - cw + nw - 1) // nw

            def issue_idx(kk, b, ecount=ecount, edges_hbm=edges_hbm, cw=cw):
                c = cw + kk * nw
                pltpu.async_copy(
                    edges_hbm.at[pl.ds(c * ecount, ecount)],
                    idx_v.at[pl.ds(b * EMAX, ecount)], isems[b])

            def wait_idx(b, ecount=ecount, edges_hbm=edges_hbm):
                pltpu.make_async_copy(
                    edges_hbm.at[pl.ds(0, ecount)],
                    idx_v.at[pl.ds(b * EMAX, ecount)], isems[b]).wait()

            def issue_self(kk, b, start=start, cw=cw):
                c = cw + kk * nw
                pltpu.async_copy(atoms_hbm.at[pl.ds(start + c * B, B)],
                                 self_v.at[b], fsems[b])

            def wait_self(b):
                pltpu.make_async_copy(atoms_hbm.at[pl.ds(0, B)],
                                      self_v.at[b], fsems[b]).wait()

            def issue_gath(b, pieces=pieces):
                for off, sz in pieces:
                    pltpu.async_copy(
                        atoms_hbm.at[idx_v.at[pl.ds(b * EMAX + off, sz)]],
                        gath_v.at[b, pl.ds(off, sz)], gsems[b])

            def wait_gath(b, ecount=ecount):
                pltpu.make_async_copy(
                    atoms_hbm.at[pl.ds(0, ecount)],
                    gath_v.at[b, pl.ds(0, ecount)], gsems[b]).wait()

            def compute(kk, b, d=d, start=start, cw=cw):
                c = cw + kk * nw

                @pl.when(kk >= 2)
                def _():
                    pltpu.make_async_copy(
                        outb_v.at[b], out_hbm.at[pl.ds(0, B)], ssems[b]).wait()

                wait_self(b)

                @pl.loop(0, B)
                def row_body(i):
                    base = i * d
                    accs = [self_v[b, i, pl.ds(s * 16, 16)]
                            for s in range(D // 16)]
                    for j in range(d):
                        for s in range(D // 16):
                            accs[s] = jnp.maximum(
                                accs[s], gath_v[b, base + j, pl.ds(s * 16, 16)])
                    for s in range(D // 16):
                        outb_v[b, i, pl.ds(s * 16, 16)] = accs[s]

                pltpu.async_copy(outb_v.at[b],
                                 out_hbm.at[pl.ds(start + c * B, B)], ssems[b])

            issue_idx(1, 1)
            issue_self(0, 0)
            issue_self(1, 1)
            if d == 1:
                pltpu.make_async_copy(
                    a1.at[pl.ds(0, ptotal)],
                    pidx_v.at[pl.ds(0, ptotal)], prosem).wait()
            pltpu.async_copy(
                atoms_hbm.at[pidx_v.at[pl.ds(20 * d * (d - 1), ecount)]],
                gath_v.at[0, pl.ds(0, ecount)], gsems[0])

            @pl.loop(0, nch_w)
            def chunk_body(kk, nch_w=nch_w, issue_idx=issue_idx,
                           wait_idx=wait_idx, issue_gath=issue_gath,
                           wait_gath=wait_gath, issue_self=issue_self,
                           compute=compute):
                for b in (0, 1):
                    @pl.when(kk % 2 == b)
                    def _(b=b, kk=kk):
                        wait_gath(b)

                        @pl.when(kk + 2 < nch_w)
                        def _():
                            issue_idx(kk + 2, b)

                        @pl.when(kk + 1 < nch_w)
                        def _(b=b):
                            wait_idx(1 - b)
                            issue_gath(1 - b)

                        compute(kk, b)

                        @pl.when(kk + 2 < nch_w)
                        def _():
                            issue_self(kk + 2, b)

            for b in (0, 1):
                pltpu.make_async_copy(
                    outb_v.at[b], out_hbm.at[pl.ds(0, B)], ssems[b]).wait()

        @pl.when(wid < 2)
        def _():
            pltpu.make_async_copy(atoms_hbm.at[pl.ds(0, 320)],
                                  out_hbm.at[pl.ds(0, 320)], dsem).wait()

        @pl.when(wid >= 2)
        def _():
            pltpu.make_async_copy(atoms_hbm.at[pl.ds(0, 312)],
                                  out_hbm.at[pl.ds(0, 312)], dsem).wait()

    return k(atoms, *adj_flats)


def kernel(atoms, deg_slice, membership, deg_adj_1, deg_adj_2, deg_adj_3,
           deg_adj_4, deg_adj_5, deg_adj_6, deg_adj_7, deg_adj_8,
           deg_adj_9, deg_adj_10):
    del deg_slice, membership
    adj_flats = [a.reshape(-1) for a in
                 (deg_adj_1, deg_adj_2, deg_adj_3, deg_adj_4, deg_adj_5,
                  deg_adj_6, deg_adj_7, deg_adj_8, deg_adj_9, deg_adj_10)]
    return _pool(atoms, adj_flats)
